# Initial kernel scaffold; baseline (speedup 1.0000x reference)
import jax
import jax.numpy as jnp
from jax.experimental import pallas as pl


def kernel(seq_indices, token_embed, pos_embed):
    raise NotImplementedError("write your pallas kernel here")


# speedup vs baseline: 3.5406x; 3.5406x over previous
import jax
import jax.numpy as jnp
from jax.experimental import pallas as pl

_NUM_TOKENS = 5
_BL = 512


def _embed_kernel(idx_ref, tok_ref, pos_ref, out_ref):
    idx = idx_ref[...]
    acc = pos_ref[...]
    for t in range(1, _NUM_TOKENS):
        row = tok_ref[pl.ds(t, 1), :]
        acc = acc + jnp.where(idx == t, row, 0.0)
    out_ref[...] = acc


def kernel(seq_indices, token_embed, pos_embed):
    B, L = seq_indices.shape
    E = token_embed.shape[1]
    idx2d = seq_indices.astype(jnp.int32).reshape(B * L, 1)
    jl = L // _BL
    grid = (B, jl)
    out = pl.pallas_call(
        _embed_kernel,
        grid=grid,
        in_specs=[
            pl.BlockSpec((_BL, 1), lambda i, j: (i * jl + j, 0)),
            pl.BlockSpec((_NUM_TOKENS, E), lambda i, j: (0, 0)),
            pl.BlockSpec((_BL, E), lambda i, j: (j, 0)),
        ],
        out_specs=pl.BlockSpec((_BL, E), lambda i, j: (i * jl + j, 0)),
        out_shape=jax.ShapeDtypeStruct((B * L, E), jnp.float32),
    )(idx2d, token_embed, pos_embed)
    return out.reshape(B, L, E)
